# baseline (speedup 1.0000x reference)
import functools

import jax
import jax.numpy as jnp
from jax import lax
from jax.experimental import pallas as pl
from jax.experimental.pallas import tpu as pltpu
from jax.experimental.pallas import tpu_sc as plsc

NN = 10000
EE = 160000
D = 256
NB_BATCH = 16
Np = 10240
NT = 16
EPT = EE // NT
KC = 80
NKC = EPT // KC
NF = 4
FW = D // NF
BN = 2048
BE = 3200
F32 = jnp.float32
I32 = jnp.int32


def _prelu(x, p):
    return jnp.where(x >= 0, x, p * x)


def _store_heads(xpT_ref, ss_ref, sd_ref, xb, w, a_s_, a_d_):
    for h in range(2):
        xph = lax.dot_general(xb, w[:, h * D:(h + 1) * D],
                              (((1,), (0,)), ((), ())),
                              preferred_element_type=F32)
        for f in range(NF):
            xpT_ref[h, f] = xph[:, f * FW:(f + 1) * FW]
        ss_ref[h] = lax.dot_general(xph, a_s_[h:h + 1, :],
                                    (((1,), (1,)), ((), ())),
                                    preferred_element_type=F32)[:, 0]
        sd_ref[h] = lax.dot_general(xph, a_d_[h:h + 1, :],
                                    (((1,), (1,)), ((), ())),
                                    preferred_element_type=F32)[:, 0]


def _node_prep(x2d, W, a_s, a_d, cin):
    def body(x_ref, w_ref, as_ref, ad_ref, xpT_ref, ss_ref, sd_ref):
        _store_heads(xpT_ref, ss_ref, sd_ref,
                     x_ref[...], w_ref[...], as_ref[...], ad_ref[...])
    return pl.pallas_call(
        body,
        grid=(Np // BN,),
        in_specs=[pl.BlockSpec((BN, cin), lambda i: (i, 0)),
                  pl.BlockSpec((cin, 2 * D), lambda i: (0, 0)),
                  pl.BlockSpec((2, D), lambda i: (0, 0)),
                  pl.BlockSpec((2, D), lambda i: (0, 0))],
        out_specs=[pl.BlockSpec((2, NF, BN, FW), lambda i: (0, 0, i, 0)),
                   pl.BlockSpec((2, BN), lambda i: (0, i)),
                   pl.BlockSpec((2, BN), lambda i: (0, i))],
        out_shape=[jax.ShapeDtypeStruct((2, NF, Np, FW), F32),
                   jax.ShapeDtypeStruct((2, Np), F32),
                   jax.ShapeDtypeStruct((2, Np), F32)],
    )(x2d, W, a_s, a_d)


def _head_mean(o_ref, b_ref, p_ref):
    parts = [0.5 * (o_ref[0, f] + o_ref[1, f]) for f in range(NF)]
    hm = jnp.concatenate(parts, axis=1) + b_ref[...]
    return _prelu(hm, p_ref[0, 0])


def _mid_prep(o, b, p, W, a_s, a_d):
    def body(o_ref, b_ref, p_ref, w_ref, as_ref, ad_ref,
             xpT_ref, ss_ref, sd_ref):
        hb = _head_mean(o_ref, b_ref, p_ref)
        _store_heads(xpT_ref, ss_ref, sd_ref,
                     hb, w_ref[...], as_ref[...], ad_ref[...])
    return pl.pallas_call(
        body,
        grid=(Np // BN,),
        in_specs=[pl.BlockSpec((2, NF, BN, FW), lambda i: (0, 0, i, 0)),
                  pl.BlockSpec((1, D), lambda i: (0, 0)),
                  pl.BlockSpec((1, 1), lambda i: (0, 0)),
                  pl.BlockSpec((D, 2 * D), lambda i: (0, 0)),
                  pl.BlockSpec((2, D), lambda i: (0, 0)),
                  pl.BlockSpec((2, D), lambda i: (0, 0))],
        out_specs=[pl.BlockSpec((2, NF, BN, FW), lambda i: (0, 0, i, 0)),
                   pl.BlockSpec((2, BN), lambda i: (0, i)),
                   pl.BlockSpec((2, BN), lambda i: (0, i))],
        out_shape=[jax.ShapeDtypeStruct((2, NF, Np, FW), F32),
                   jax.ShapeDtypeStruct((2, Np), F32),
                   jax.ShapeDtypeStruct((2, Np), F32)],
    )(o, b, p, W, a_s, a_d)


def _edge_prep(ea, We1, ae1, We2, ae2):
    def body(ea_ref, we1_ref, ae1_ref, we2_ref, ae2_ref, se1_ref, se2_ref):
        eb = ea_ref[...]
        for h in range(2):
            ve1 = lax.dot_general(we1_ref[:, h * D:(h + 1) * D],
                                  ae1_ref[h:h + 1, :],
                                  (((1,), (1,)), ((), ())),
                                  preferred_element_type=F32)
            se1_ref[h] = lax.dot_general(eb, ve1, (((1,), (0,)), ((), ())),
                                         preferred_element_type=F32)[:, 0]
            ve2 = lax.dot_general(we2_ref[:, h * D:(h + 1) * D],
                                  ae2_ref[h:h + 1, :],
                                  (((1,), (1,)), ((), ())),
                                  preferred_element_type=F32)
            se2_ref[h] = lax.dot_general(eb, ve2, (((1,), (0,)), ((), ())),
                                         preferred_element_type=F32)[:, 0]
    cin = 128
    return pl.pallas_call(
        body,
        grid=(EE // BE,),
        in_specs=[pl.BlockSpec((BE, cin), lambda i: (i, 0)),
                  pl.BlockSpec((cin, 2 * D), lambda i: (0, 0)),
                  pl.BlockSpec((2, D), lambda i: (0, 0)),
                  pl.BlockSpec((cin, 2 * D), lambda i: (0, 0)),
                  pl.BlockSpec((2, D), lambda i: (0, 0))],
        out_specs=[pl.BlockSpec((2, BE), lambda i: (0, i)),
                   pl.BlockSpec((2, BE), lambda i: (0, i))],
        out_shape=[jax.ShapeDtypeStruct((2, EE), F32),
                   jax.ShapeDtypeStruct((2, EE), F32)],
    )(ea, We1, ae1, We2, ae2)


def _final_pool(o, b2, p2, G1, gb1, gp1, G2, gb2, gp2, G3p, gb3, batch3):
    nsteps = Np // BN

    def body(o_ref, bt_ref, b2_ref, p2_ref, g1_ref, gb1_ref, gp1_ref,
             g2_ref, gb2_ref, gp2_ref, g3_ref, gb3_ref, out_ref, pnum, gd):
        i = pl.program_id(0)

        @pl.when(i == 0)
        def _():
            pnum[...] = jnp.zeros_like(pnum)
            gd[...] = jnp.zeros_like(gd)

        h2 = _head_mean(o_ref, b2_ref, p2_ref)
        g = jnp.dot(h2, g1_ref[...], preferred_element_type=F32) + gb1_ref[...]
        g = _prelu(g, gp1_ref[0, 0])
        g = jnp.dot(g, g2_ref[...], preferred_element_type=F32) + gb2_ref[...]
        g = _prelu(g, gp2_ref[0, 0])
        g = jnp.dot(g, g3_ref[...], preferred_element_type=F32)
        gcol = g[:, 0:1] + gb3_ref[...]
        ge = jnp.exp(gcol)
        bt = bt_ref[0, 0, :]
        onehot = (bt[:, None] ==
                  lax.broadcasted_iota(I32, (1, NB_BATCH), 1)).astype(F32)
        gd[...] += lax.dot_general(onehot, ge, (((0,), (0,)), ((), ())),
                                   preferred_element_type=F32)
        pnum[...] += lax.dot_general(onehot, ge * h2,
                                     (((0,), (0,)), ((), ())),
                                     preferred_element_type=F32)

        @pl.when(i == nsteps - 1)
        def _():
            out_ref[...] = pnum[...] / (gd[...] + 1e-16)

    return pl.pallas_call(
        body,
        grid=(nsteps,),
        in_specs=[pl.BlockSpec((2, NF, BN, FW), lambda i: (0, 0, i, 0)),
                  pl.BlockSpec((1, 1, BN), lambda i: (i, 0, 0)),
                  pl.BlockSpec((1, D), lambda i: (0, 0)),
                  pl.BlockSpec((1, 1), lambda i: (0, 0)),
                  pl.BlockSpec((D, D), lambda i: (0, 0)),
                  pl.BlockSpec((1, D), lambda i: (0, 0)),
                  pl.BlockSpec((1, 1), lambda i: (0, 0)),
                  pl.BlockSpec((D, D), lambda i: (0, 0)),
                  pl.BlockSpec((1, D), lambda i: (0, 0)),
                  pl.BlockSpec((1, 1), lambda i: (0, 0)),
                  pl.BlockSpec((D, 128), lambda i: (0, 0)),
                  pl.BlockSpec((1, 1), lambda i: (0, 0))],
        out_specs=pl.BlockSpec((NB_BATCH, D), lambda i: (0, 0)),
        out_shape=jax.ShapeDtypeStruct((NB_BATCH, D), F32),
        scratch_shapes=[pltpu.VMEM((NB_BATCH, D), F32),
                        pltpu.VMEM((NB_BATCH, 1), F32)],
    )(o, batch3, b2, p2, G1, gb1, gp1, G2, gb2, gp2, G3p, gb3)


_sc_mesh = plsc.VectorSubcoreMesh(core_axis_name="c", subcore_axis_name="s")


@functools.partial(
    pl.kernel,
    out_type=jax.ShapeDtypeStruct((2 * NF * Np, FW), F32),
    mesh=_sc_mesh,
    compiler_params=pltpu.CompilerParams(use_tc_tiling_on_sc=False,
                                         needs_layout_passes=False),
    scratch_types=[
        pltpu.VMEM((Np,), F32),
        pltpu.VMEM((Np,), F32),
        pltpu.VMEM((NKC, KC), I32),
        pltpu.VMEM((NKC, KC), I32),
        pltpu.VMEM((EPT,), F32),
        pltpu.VMEM((EPT,), F32),
        pltpu.VMEM((KC, FW), F32),
        pltpu.VMEM((KC, FW), F32),
        pltpu.VMEM((KC, 16), F32),
        pltpu.VMEM((64, FW), F32),
        pltpu.VMEM((64, 16), F32),
        pltpu.VMEM((16, FW), F32),
        pltpu.VMEM((16, 16), F32),
        pltpu.VMEM_SHARED((Np, FW), F32),
        pltpu.VMEM_SHARED((Np, 16), F32),
        pltpu.SemaphoreType.DMA,
        pltpu.SemaphoreType.DMA,
    ],
)
def _sc_gat(ss_hbm, sd_hbm, se_hbm, src_hbm, dst_hbm, xp_hbm, out_hbm,
            ssv, sdv, srcv, didxv, sev, wv, rowsa, rowsb, wbuf,
            tmp, tmpd, zbuf, zbufd, accsh, densh, sema, semb):
    c = lax.axis_index("c")
    s = lax.axis_index("s")
    ii = lax.iota(I32, 16)
    npt = Np // NT

    pltpu.sync_copy(ss_hbm.at[pl.ds(c * Np, Np)], ssv)
    pltpu.sync_copy(sd_hbm.at[pl.ds(c * Np, Np)], sdv)
    pltpu.sync_copy(src_hbm.at[s], srcv)
    pltpu.sync_copy(dst_hbm.at[s], didxv)
    pltpu.sync_copy(se_hbm.at[pl.ds(c * EE + s * EPT, EPT)], sev)

    z16 = jnp.zeros((16,), F32)
    for r in range(16):
        for q in range(FW // 16):
            zbuf[r, pl.ds(q * 16, 16)] = z16
        zbufd[r] = z16

    def logit_body(j, _):
        for k in range(KC // 16):
            sl = pl.ds(k * 16, 16)
            sr = srcv[j, sl]
            dt = didxv[j, sl]
            a = (plsc.load_gather(ssv, [sr]) + plsc.load_gather(sdv, [dt])
                 + sev[pl.ds(j * KC + k * 16, 16)])
            a = jnp.where(a >= 0, a, 0.2 * a)
            wv[pl.ds(j * KC + k * 16, 16)] = jnp.exp(a)
            srcv[j, sl] = sr + c * (NF * Np)
        return 0
    lax.fori_loop(0, NKC, logit_body, 0)

    zz = jnp.zeros((16,), I32)

    def pass_body(p, _):
        @pl.when(p > 0)
        def _():
            def adv_body(j, _):
                for k in range(KC // 16):
                    sl = pl.ds(k * 16, 16)
                    srcv[j, sl] = srcv[j, sl] + Np
                return 0
            lax.fori_loop(0, NKC, adv_body, 0)

        def zero_body(k, _):
            pltpu.sync_copy(zbuf, accsh.at[pl.ds(s * npt + k * 16, 16)])
            return 0
        lax.fori_loop(0, npt // 16, zero_body, 0)

        @pl.when(p == 0)
        def _():
            def zden_body(k, _):
                pltpu.sync_copy(zbufd, densh.at[pl.ds(s * npt + k * 16, 16)])
                return 0
            lax.fori_loop(0, npt // 16, zden_body, 0)
        plsc.subcore_barrier()

        def process(j, rows):
            def grp_body(g, _):
                rbase = g * 16
                for r in range(16):
                    av = plsc.load_gather(
                        wv, [jnp.full((16,), j * KC + rbase + r, I32)])
                    wbuf[rbase + r] = av
                    for q in range(FW // 16):
                        sl = pl.ds(q * 16, 16)
                        rows[rbase + r, sl] = rows[rbase + r, sl] * av
                return 0
            lax.fori_loop(0, KC // 16, grp_body, 0)
            pltpu.sync_copy(rows, accsh.at[didxv.at[j]], add=True)

            @pl.when(p == 0)
            def _():
                pltpu.sync_copy(wbuf, densh.at[didxv.at[j]], add=True)

        def start_gather(j, rows, sem):
            pltpu.async_copy(xp_hbm.at[srcv.at[j]], rows, sem)

        def wait_gather(rows, sem):
            pltpu.make_async_copy(xp_hbm.at[srcv.at[0]], rows, sem).wait()

        start_gather(0, rowsa, sema)

        def pair_body(t, _):
            j0 = 2 * t
            j1 = j0 + 1
            start_gather(j1, rowsb, semb)
            wait_gather(rowsa, sema)
            process(j0, rowsa)

            @pl.when(j0 + 2 < NKC)
            def _():
                start_gather(j0 + 2, rowsa, sema)
            wait_gather(rowsb, semb)
            process(j1, rowsb)
            return 0
        lax.fori_loop(0, (NKC - 1) // 2, pair_body, 0)
        wait_gather(rowsa, sema)
        process(NKC - 1, rowsa)
        plsc.subcore_barrier()

        obase = (c * NF + p) * Np + s * npt

        def co_body(k, _):
            sl = pl.ds(s * npt + k * 64, 64)
            pltpu.sync_copy(accsh.at[sl], tmp)
            pltpu.sync_copy(densh.at[sl], tmpd)
            for g in range(4):
                denv = plsc.load_gather(tmpd, [ii + g * 16, zz])
                wbuf[g] = 1.0 / (denv + 1e-16)

            def div_body(r, _):
                rcp = plsc.load_gather(
                    wbuf, [jnp.full((16,), r >> 4, I32),
                           jnp.full((16,), r & 15, I32)])
                for q in range(FW // 16):
                    qsl = pl.ds(q * 16, 16)
                    tmp[r, qsl] = tmp[r, qsl] * rcp
                return 0
            lax.fori_loop(0, 64, div_body, 0)
            pltpu.sync_copy(tmp, out_hbm.at[pl.ds(obase + k * 64, 64)])
            return 0
        lax.fori_loop(0, npt // 64, co_body, 0)
        plsc.subcore_barrier()
        return 0

    lax.fori_loop(0, NF, pass_body, 0)


def kernel(x, edge_index, edge_attr, batch,
           W1, We1, as1, ad1, ae1, b1, p1,
           W2, We2, as2, ad2, ae2, b2, p2,
           G1, gb1, gp1, G2, gb2, gp2, G3, gb3):
    src = edge_index[0]
    dst = edge_index[1]
    src3 = src.reshape(NT, NKC, KC)
    dst3 = dst.reshape(NT, NKC, KC)

    xpad = jnp.pad(x, ((0, Np - NN), (0, 0)))
    xpT1, ss1, sd1 = _node_prep(xpad, W1, as1, ad1, 128)
    se1, se2 = _edge_prep(edge_attr, We1, ae1, We2, ae2)

    out1 = _sc_gat(ss1.reshape(-1), sd1.reshape(-1), se1.reshape(-1),
                   src3, dst3, xpT1.reshape(2 * NF * Np, FW))

    xpT2, ss2, sd2 = _mid_prep(out1.reshape(2, NF, Np, FW),
                               b1.reshape(1, D), p1.reshape(1, 1),
                               W2, as2, ad2)

    out2 = _sc_gat(ss2.reshape(-1), sd2.reshape(-1), se2.reshape(-1),
                   src3, dst3, xpT2.reshape(2 * NF * Np, FW))

    batch3 = jnp.pad(batch, (0, Np - NN),
                     constant_values=NB_BATCH).reshape(Np // BN, 1, BN)
    G3p = jnp.pad(G3, ((0, 0), (0, 127)))
    pooled = _final_pool(out2.reshape(2, NF, Np, FW),
                         b2.reshape(1, D), p2.reshape(1, 1),
                         G1, gb1.reshape(1, D), gp1.reshape(1, 1),
                         G2, gb2.reshape(1, D), gp2.reshape(1, 1),
                         G3p, gb3.reshape(1, 1), batch3)
    return pooled.reshape(NB_BATCH, 2, 128)


# speedup vs baseline: 1.2543x; 1.2543x over previous
import functools

import jax
import jax.numpy as jnp
from jax import lax
from jax.experimental import pallas as pl
from jax.experimental.pallas import tpu as pltpu
from jax.experimental.pallas import tpu_sc as plsc

NN = 10000
EE = 160000
D = 256
NB_BATCH = 16
Np = 10240
NT = 16
EPT = EE // NT
KC = 80
NKC = EPT // KC
NF = 4
FW = D // NF
BN = 2048
BE = 3200
F32 = jnp.float32
I32 = jnp.int32


def _prelu(x, p):
    return jnp.where(x >= 0, x, p * x)


def _store_heads(xpT_ref, ss_ref, sd_ref, xb, w, a_s_, a_d_):
    for h in range(2):
        xph = lax.dot_general(xb, w[:, h * D:(h + 1) * D],
                              (((1,), (0,)), ((), ())),
                              preferred_element_type=F32)
        for f in range(NF):
            xpT_ref[h, f] = xph[:, f * FW:(f + 1) * FW]
        ss_ref[h] = lax.dot_general(xph, a_s_[h:h + 1, :],
                                    (((1,), (1,)), ((), ())),
                                    preferred_element_type=F32)[:, 0]
        sd_ref[h] = lax.dot_general(xph, a_d_[h:h + 1, :],
                                    (((1,), (1,)), ((), ())),
                                    preferred_element_type=F32)[:, 0]


def _node_prep(x2d, W, a_s, a_d, cin):
    def body(x_ref, w_ref, as_ref, ad_ref, xpT_ref, ss_ref, sd_ref):
        _store_heads(xpT_ref, ss_ref, sd_ref,
                     x_ref[...], w_ref[...], as_ref[...], ad_ref[...])
    return pl.pallas_call(
        body,
        grid=(Np // BN,),
        in_specs=[pl.BlockSpec((BN, cin), lambda i: (i, 0)),
                  pl.BlockSpec((cin, 2 * D), lambda i: (0, 0)),
                  pl.BlockSpec((2, D), lambda i: (0, 0)),
                  pl.BlockSpec((2, D), lambda i: (0, 0))],
        out_specs=[pl.BlockSpec((2, NF, BN, FW), lambda i: (0, 0, i, 0)),
                   pl.BlockSpec((2, BN), lambda i: (0, i)),
                   pl.BlockSpec((2, BN), lambda i: (0, i))],
        out_shape=[jax.ShapeDtypeStruct((2, NF, Np, FW), F32),
                   jax.ShapeDtypeStruct((2, Np), F32),
                   jax.ShapeDtypeStruct((2, Np), F32)],
    )(x2d, W, a_s, a_d)


def _head_mean(o_ref, b_ref, p_ref):
    parts = [0.5 * (o_ref[0, f] + o_ref[1, f]) for f in range(NF)]
    hm = jnp.concatenate(parts, axis=1) + b_ref[...]
    return _prelu(hm, p_ref[0, 0])


def _mid_prep(o, b, p, W, a_s, a_d):
    def body(o_ref, b_ref, p_ref, w_ref, as_ref, ad_ref,
             xpT_ref, ss_ref, sd_ref):
        hb = _head_mean(o_ref, b_ref, p_ref)
        _store_heads(xpT_ref, ss_ref, sd_ref,
                     hb, w_ref[...], as_ref[...], ad_ref[...])
    return pl.pallas_call(
        body,
        grid=(Np // BN,),
        in_specs=[pl.BlockSpec((2, NF, BN, FW), lambda i: (0, 0, i, 0)),
                  pl.BlockSpec((1, D), lambda i: (0, 0)),
                  pl.BlockSpec((1, 1), lambda i: (0, 0)),
                  pl.BlockSpec((D, 2 * D), lambda i: (0, 0)),
                  pl.BlockSpec((2, D), lambda i: (0, 0)),
                  pl.BlockSpec((2, D), lambda i: (0, 0))],
        out_specs=[pl.BlockSpec((2, NF, BN, FW), lambda i: (0, 0, i, 0)),
                   pl.BlockSpec((2, BN), lambda i: (0, i)),
                   pl.BlockSpec((2, BN), lambda i: (0, i))],
        out_shape=[jax.ShapeDtypeStruct((2, NF, Np, FW), F32),
                   jax.ShapeDtypeStruct((2, Np), F32),
                   jax.ShapeDtypeStruct((2, Np), F32)],
    )(o, b, p, W, a_s, a_d)


def _edge_prep(ea, We1, ae1, We2, ae2):
    def body(ea_ref, we1_ref, ae1_ref, we2_ref, ae2_ref, se1_ref, se2_ref):
        eb = ea_ref[...]
        for h in range(2):
            ve1 = lax.dot_general(we1_ref[:, h * D:(h + 1) * D],
                                  ae1_ref[h:h + 1, :],
                                  (((1,), (1,)), ((), ())),
                                  preferred_element_type=F32)
            se1_ref[h] = lax.dot_general(eb, ve1, (((1,), (0,)), ((), ())),
                                         preferred_element_type=F32)[:, 0]
            ve2 = lax.dot_general(we2_ref[:, h * D:(h + 1) * D],
                                  ae2_ref[h:h + 1, :],
                                  (((1,), (1,)), ((), ())),
                                  preferred_element_type=F32)
            se2_ref[h] = lax.dot_general(eb, ve2, (((1,), (0,)), ((), ())),
                                         preferred_element_type=F32)[:, 0]
    cin = 128
    return pl.pallas_call(
        body,
        grid=(EE // BE,),
        in_specs=[pl.BlockSpec((BE, cin), lambda i: (i, 0)),
                  pl.BlockSpec((cin, 2 * D), lambda i: (0, 0)),
                  pl.BlockSpec((2, D), lambda i: (0, 0)),
                  pl.BlockSpec((cin, 2 * D), lambda i: (0, 0)),
                  pl.BlockSpec((2, D), lambda i: (0, 0))],
        out_specs=[pl.BlockSpec((2, BE), lambda i: (0, i)),
                   pl.BlockSpec((2, BE), lambda i: (0, i))],
        out_shape=[jax.ShapeDtypeStruct((2, EE), F32),
                   jax.ShapeDtypeStruct((2, EE), F32)],
    )(ea, We1, ae1, We2, ae2)


def _final_pool(o, b2, p2, G1, gb1, gp1, G2, gb2, gp2, G3p, gb3, batch3):
    nsteps = Np // BN

    def body(o_ref, bt_ref, b2_ref, p2_ref, g1_ref, gb1_ref, gp1_ref,
             g2_ref, gb2_ref, gp2_ref, g3_ref, gb3_ref, out_ref, pnum, gd):
        i = pl.program_id(0)

        @pl.when(i == 0)
        def _():
            pnum[...] = jnp.zeros_like(pnum)
            gd[...] = jnp.zeros_like(gd)

        h2 = _head_mean(o_ref, b2_ref, p2_ref)
        g = jnp.dot(h2, g1_ref[...], preferred_element_type=F32) + gb1_ref[...]
        g = _prelu(g, gp1_ref[0, 0])
        g = jnp.dot(g, g2_ref[...], preferred_element_type=F32) + gb2_ref[...]
        g = _prelu(g, gp2_ref[0, 0])
        g = jnp.dot(g, g3_ref[...], preferred_element_type=F32)
        gcol = g[:, 0:1] + gb3_ref[...]
        ge = jnp.exp(gcol)
        bt = bt_ref[0, 0, :]
        onehot = (bt[:, None] ==
                  lax.broadcasted_iota(I32, (1, NB_BATCH), 1)).astype(F32)
        gd[...] += lax.dot_general(onehot, ge, (((0,), (0,)), ((), ())),
                                   preferred_element_type=F32)
        pnum[...] += lax.dot_general(onehot, ge * h2,
                                     (((0,), (0,)), ((), ())),
                                     preferred_element_type=F32)

        @pl.when(i == nsteps - 1)
        def _():
            out_ref[...] = pnum[...] / (gd[...] + 1e-16)

    return pl.pallas_call(
        body,
        grid=(nsteps,),
        in_specs=[pl.BlockSpec((2, NF, BN, FW), lambda i: (0, 0, i, 0)),
                  pl.BlockSpec((1, 1, BN), lambda i: (i, 0, 0)),
                  pl.BlockSpec((1, D), lambda i: (0, 0)),
                  pl.BlockSpec((1, 1), lambda i: (0, 0)),
                  pl.BlockSpec((D, D), lambda i: (0, 0)),
                  pl.BlockSpec((1, D), lambda i: (0, 0)),
                  pl.BlockSpec((1, 1), lambda i: (0, 0)),
                  pl.BlockSpec((D, D), lambda i: (0, 0)),
                  pl.BlockSpec((1, D), lambda i: (0, 0)),
                  pl.BlockSpec((1, 1), lambda i: (0, 0)),
                  pl.BlockSpec((D, 128), lambda i: (0, 0)),
                  pl.BlockSpec((1, 1), lambda i: (0, 0))],
        out_specs=pl.BlockSpec((NB_BATCH, D), lambda i: (0, 0)),
        out_shape=jax.ShapeDtypeStruct((NB_BATCH, D), F32),
        scratch_shapes=[pltpu.VMEM((NB_BATCH, D), F32),
                        pltpu.VMEM((NB_BATCH, 1), F32)],
    )(o, batch3, b2, p2, G1, gb1, gp1, G2, gb2, gp2, G3p, gb3)


_sc_mesh = plsc.VectorSubcoreMesh(core_axis_name="c", subcore_axis_name="s")


@functools.partial(
    pl.kernel,
    out_type=jax.ShapeDtypeStruct((2 * NF * Np, FW), F32),
    mesh=_sc_mesh,
    compiler_params=pltpu.CompilerParams(use_tc_tiling_on_sc=False,
                                         needs_layout_passes=False),
    scratch_types=[
        pltpu.VMEM((Np,), F32),
        pltpu.VMEM((Np,), F32),
        pltpu.VMEM((NKC, KC), I32),
        pltpu.VMEM((NKC, KC), I32),
        pltpu.VMEM((EPT,), F32),
        pltpu.VMEM((EPT,), F32),
        pltpu.VMEM((KC, FW), F32),
        pltpu.VMEM((KC, FW), F32),
        pltpu.VMEM((KC, 16), F32),
        pltpu.VMEM((64, FW), F32),
        pltpu.VMEM((64, 16), F32),
        pltpu.VMEM((16, FW), F32),
        pltpu.VMEM((16, 16), F32),
        pltpu.VMEM_SHARED((Np, FW), F32),
        pltpu.VMEM_SHARED((Np, 16), F32),
        pltpu.SemaphoreType.DMA,
        pltpu.SemaphoreType.DMA,
    ],
)
def _sc_gat(ss_hbm, sd_hbm, se_hbm, src_hbm, dst_hbm, xp_hbm, out_hbm,
            ssv, sdv, srcv, didxv, sev, wv, rowsa, rowsb, wbuf,
            tmp, tmpd, zbuf, zbufd, accsh, densh, sema, semb):
    c = lax.axis_index("c")
    s = lax.axis_index("s")
    ii = lax.iota(I32, 16)
    npt = Np // NT

    pltpu.sync_copy(ss_hbm.at[pl.ds(c * Np, Np)], ssv)
    pltpu.sync_copy(sd_hbm.at[pl.ds(c * Np, Np)], sdv)
    pltpu.sync_copy(src_hbm.at[s], srcv)
    pltpu.sync_copy(dst_hbm.at[s], didxv)
    pltpu.sync_copy(se_hbm.at[pl.ds(c * EE + s * EPT, EPT)], sev)

    z16 = jnp.zeros((16,), F32)
    for r in range(16):
        for q in range(FW // 16):
            zbuf[r, pl.ds(q * 16, 16)] = z16
        zbufd[r] = z16

    def logit_body(j, _):
        for k in range(KC // 16):
            sl = pl.ds(k * 16, 16)
            sr = srcv[j, sl]
            dt = didxv[j, sl]
            a = (plsc.load_gather(ssv, [sr]) + plsc.load_gather(sdv, [dt])
                 + sev[pl.ds(j * KC + k * 16, 16)])
            a = jnp.where(a >= 0, a, 0.2 * a)
            wv[pl.ds(j * KC + k * 16, 16)] = jnp.exp(a)
            srcv[j, sl] = sr + c * (NF * Np)
        return 0
    lax.fori_loop(0, NKC, logit_body, 0)

    zz = jnp.zeros((16,), I32)

    def pass_body(p, _):
        @pl.when(p > 0)
        def _():
            def adv_body(j, _):
                for k in range(KC // 16):
                    sl = pl.ds(k * 16, 16)
                    srcv[j, sl] = srcv[j, sl] + Np
                return 0
            lax.fori_loop(0, NKC, adv_body, 0)

        def zero_body(k, _):
            pltpu.sync_copy(zbuf, accsh.at[pl.ds(s * npt + k * 16, 16)])
            return 0
        lax.fori_loop(0, npt // 16, zero_body, 0)

        @pl.when(p == 0)
        def _():
            def zden_body(k, _):
                pltpu.sync_copy(zbufd, densh.at[pl.ds(s * npt + k * 16, 16)])
                return 0
            lax.fori_loop(0, npt // 16, zden_body, 0)
        plsc.subcore_barrier()

        def process(j, rows):
            def row_body(r, _):
                av = plsc.load_gather(wv, [jnp.full((16,), j * KC + r, I32)])
                wbuf[r] = av
                for q in range(FW // 16):
                    sl = pl.ds(q * 16, 16)
                    rows[r, sl] = rows[r, sl] * av
                return 0
            lax.fori_loop(0, KC, row_body, 0)
            pltpu.sync_copy(rows, accsh.at[didxv.at[j]], add=True)

            @pl.when(p == 0)
            def _():
                pltpu.sync_copy(wbuf, densh.at[didxv.at[j]], add=True)

        def start_gather(j, rows, sem):
            pltpu.async_copy(xp_hbm.at[srcv.at[j]], rows, sem)

        def wait_gather(rows, sem):
            pltpu.make_async_copy(xp_hbm.at[srcv.at[0]], rows, sem).wait()

        start_gather(0, rowsa, sema)

        def pair_body(t, _):
            j0 = 2 * t
            j1 = j0 + 1
            start_gather(j1, rowsb, semb)
            wait_gather(rowsa, sema)
            process(j0, rowsa)

            @pl.when(j0 + 2 < NKC)
            def _():
                start_gather(j0 + 2, rowsa, sema)
            wait_gather(rowsb, semb)
            process(j1, rowsb)
            return 0
        lax.fori_loop(0, (NKC - 1) // 2, pair_body, 0)
        wait_gather(rowsa, sema)
        process(NKC - 1, rowsa)
        plsc.subcore_barrier()

        obase = (c * NF + p) * Np + s * npt

        def co_body(k, _):
            sl = pl.ds(s * npt + k * 64, 64)
            pltpu.sync_copy(accsh.at[sl], tmp)
            pltpu.sync_copy(densh.at[sl], tmpd)
            for g in range(4):
                denv = plsc.load_gather(tmpd, [ii + g * 16, zz])
                wbuf[g] = 1.0 / (denv + 1e-16)

            def div_body(r, _):
                rcp = plsc.load_gather(
                    wbuf, [jnp.full((16,), r >> 4, I32),
                           jnp.full((16,), r & 15, I32)])
                for q in range(FW // 16):
                    qsl = pl.ds(q * 16, 16)
                    tmp[r, qsl] = tmp[r, qsl] * rcp
                return 0
            lax.fori_loop(0, 64, div_body, 0)
            pltpu.sync_copy(tmp, out_hbm.at[pl.ds(obase + k * 64, 64)])
            return 0
        lax.fori_loop(0, npt // 64, co_body, 0)
        plsc.subcore_barrier()
        return 0

    lax.fori_loop(0, NF, pass_body, 0)


def kernel(x, edge_index, edge_attr, batch,
           W1, We1, as1, ad1, ae1, b1, p1,
           W2, We2, as2, ad2, ae2, b2, p2,
           G1, gb1, gp1, G2, gb2, gp2, G3, gb3):
    src = edge_index[0]
    dst = edge_index[1]
    src3 = src.reshape(NT, NKC, KC)
    dst3 = dst.reshape(NT, NKC, KC)

    xpad = jnp.pad(x, ((0, Np - NN), (0, 0)))
    xpT1, ss1, sd1 = _node_prep(xpad, W1, as1, ad1, 128)
    se1, se2 = _edge_prep(edge_attr, We1, ae1, We2, ae2)

    out1 = _sc_gat(ss1.reshape(-1), sd1.reshape(-1), se1.reshape(-1),
                   src3, dst3, xpT1.reshape(2 * NF * Np, FW))

    xpT2, ss2, sd2 = _mid_prep(out1.reshape(2, NF, Np, FW),
                               b1.reshape(1, D), p1.reshape(1, 1),
                               W2, as2, ad2)

    out2 = _sc_gat(ss2.reshape(-1), sd2.reshape(-1), se2.reshape(-1),
                   src3, dst3, xpT2.reshape(2 * NF * Np, FW))

    batch3 = jnp.pad(batch, (0, Np - NN),
                     constant_values=NB_BATCH).reshape(Np // BN, 1, BN)
    G3p = jnp.pad(G3, ((0, 0), (0, 127)))
    pooled = _final_pool(out2.reshape(2, NF, Np, FW),
                         b2.reshape(1, D), p2.reshape(1, 1),
                         G1, gb1.reshape(1, D), gp1.reshape(1, 1),
                         G2, gb2.reshape(1, D), gp2.reshape(1, 1),
                         G3p, gb3.reshape(1, 1), batch3)
    return pooled.reshape(NB_BATCH, 2, 128)


# speedup vs baseline: 1.2647x; 1.0083x over previous
import functools

import jax
import jax.numpy as jnp
from jax import lax
from jax.experimental import pallas as pl
from jax.experimental.pallas import tpu as pltpu
from jax.experimental.pallas import tpu_sc as plsc

NN = 10000
EE = 160000
D = 256
NB_BATCH = 16
Np = 10240
NT = 16
EPT = EE // NT
KC = 80
NKC = EPT // KC
NF = 4
FW = D // NF
BN = 2048
BE = 3200
F32 = jnp.float32
I32 = jnp.int32


def _prelu(x, p):
    return jnp.where(x >= 0, x, p * x)


def _store_heads(xpT_ref, ss_ref, sd_ref, xb, w, a_s_, a_d_):
    for h in range(2):
        xph = lax.dot_general(xb, w[:, h * D:(h + 1) * D],
                              (((1,), (0,)), ((), ())),
                              preferred_element_type=F32)
        for f in range(NF):
            xpT_ref[h, f] = xph[:, f * FW:(f + 1) * FW]
        ss_ref[h] = lax.dot_general(xph, a_s_[h:h + 1, :],
                                    (((1,), (1,)), ((), ())),
                                    preferred_element_type=F32)[:, 0]
        sd_ref[h] = lax.dot_general(xph, a_d_[h:h + 1, :],
                                    (((1,), (1,)), ((), ())),
                                    preferred_element_type=F32)[:, 0]


def _node_prep(x2d, W, a_s, a_d, cin):
    def body(x_ref, w_ref, as_ref, ad_ref, xpT_ref, ss_ref, sd_ref):
        _store_heads(xpT_ref, ss_ref, sd_ref,
                     x_ref[...], w_ref[...], as_ref[...], ad_ref[...])
    return pl.pallas_call(
        body,
        grid=(Np // BN,),
        in_specs=[pl.BlockSpec((BN, cin), lambda i: (i, 0)),
                  pl.BlockSpec((cin, 2 * D), lambda i: (0, 0)),
                  pl.BlockSpec((2, D), lambda i: (0, 0)),
                  pl.BlockSpec((2, D), lambda i: (0, 0))],
        out_specs=[pl.BlockSpec((2, NF, BN, FW), lambda i: (0, 0, i, 0)),
                   pl.BlockSpec((2, BN), lambda i: (0, i)),
                   pl.BlockSpec((2, BN), lambda i: (0, i))],
        out_shape=[jax.ShapeDtypeStruct((2, NF, Np, FW), F32),
                   jax.ShapeDtypeStruct((2, Np), F32),
                   jax.ShapeDtypeStruct((2, Np), F32)],
    )(x2d, W, a_s, a_d)


def _head_mean(o_ref, b_ref, p_ref):
    parts = [0.5 * (o_ref[0, f] + o_ref[1, f]) for f in range(NF)]
    hm = jnp.concatenate(parts, axis=1) + b_ref[...]
    return _prelu(hm, p_ref[0, 0])


def _mid_prep(o, b, p, W, a_s, a_d):
    def body(o_ref, b_ref, p_ref, w_ref, as_ref, ad_ref,
             xpT_ref, ss_ref, sd_ref):
        hb = _head_mean(o_ref, b_ref, p_ref)
        _store_heads(xpT_ref, ss_ref, sd_ref,
                     hb, w_ref[...], as_ref[...], ad_ref[...])
    return pl.pallas_call(
        body,
        grid=(Np // BN,),
        in_specs=[pl.BlockSpec((2, NF, BN, FW), lambda i: (0, 0, i, 0)),
                  pl.BlockSpec((1, D), lambda i: (0, 0)),
                  pl.BlockSpec((1, 1), lambda i: (0, 0)),
                  pl.BlockSpec((D, 2 * D), lambda i: (0, 0)),
                  pl.BlockSpec((2, D), lambda i: (0, 0)),
                  pl.BlockSpec((2, D), lambda i: (0, 0))],
        out_specs=[pl.BlockSpec((2, NF, BN, FW), lambda i: (0, 0, i, 0)),
                   pl.BlockSpec((2, BN), lambda i: (0, i)),
                   pl.BlockSpec((2, BN), lambda i: (0, i))],
        out_shape=[jax.ShapeDtypeStruct((2, NF, Np, FW), F32),
                   jax.ShapeDtypeStruct((2, Np), F32),
                   jax.ShapeDtypeStruct((2, Np), F32)],
    )(o, b, p, W, a_s, a_d)


def _edge_prep(ea, We1, ae1, We2, ae2):
    def body(ea_ref, we1_ref, ae1_ref, we2_ref, ae2_ref, se1_ref, se2_ref):
        eb = ea_ref[...]
        for h in range(2):
            ve1 = lax.dot_general(we1_ref[:, h * D:(h + 1) * D],
                                  ae1_ref[h:h + 1, :],
                                  (((1,), (1,)), ((), ())),
                                  preferred_element_type=F32)
            se1_ref[h] = lax.dot_general(eb, ve1, (((1,), (0,)), ((), ())),
                                         preferred_element_type=F32)[:, 0]
            ve2 = lax.dot_general(we2_ref[:, h * D:(h + 1) * D],
                                  ae2_ref[h:h + 1, :],
                                  (((1,), (1,)), ((), ())),
                                  preferred_element_type=F32)
            se2_ref[h] = lax.dot_general(eb, ve2, (((1,), (0,)), ((), ())),
                                         preferred_element_type=F32)[:, 0]
    cin = 128
    return pl.pallas_call(
        body,
        grid=(EE // BE,),
        in_specs=[pl.BlockSpec((BE, cin), lambda i: (i, 0)),
                  pl.BlockSpec((cin, 2 * D), lambda i: (0, 0)),
                  pl.BlockSpec((2, D), lambda i: (0, 0)),
                  pl.BlockSpec((cin, 2 * D), lambda i: (0, 0)),
                  pl.BlockSpec((2, D), lambda i: (0, 0))],
        out_specs=[pl.BlockSpec((2, BE), lambda i: (0, i)),
                   pl.BlockSpec((2, BE), lambda i: (0, i))],
        out_shape=[jax.ShapeDtypeStruct((2, EE), F32),
                   jax.ShapeDtypeStruct((2, EE), F32)],
    )(ea, We1, ae1, We2, ae2)


def _final_pool(o, b2, p2, G1, gb1, gp1, G2, gb2, gp2, G3p, gb3, batch3):
    nsteps = Np // BN

    def body(o_ref, bt_ref, b2_ref, p2_ref, g1_ref, gb1_ref, gp1_ref,
             g2_ref, gb2_ref, gp2_ref, g3_ref, gb3_ref, out_ref, pnum, gd):
        i = pl.program_id(0)

        @pl.when(i == 0)
        def _():
            pnum[...] = jnp.zeros_like(pnum)
            gd[...] = jnp.zeros_like(gd)

        h2 = _head_mean(o_ref, b2_ref, p2_ref)
        g = jnp.dot(h2, g1_ref[...], preferred_element_type=F32) + gb1_ref[...]
        g = _prelu(g, gp1_ref[0, 0])
        g = jnp.dot(g, g2_ref[...], preferred_element_type=F32) + gb2_ref[...]
        g = _prelu(g, gp2_ref[0, 0])
        g = jnp.dot(g, g3_ref[...], preferred_element_type=F32)
        gcol = g[:, 0:1] + gb3_ref[...]
        ge = jnp.exp(gcol)
        bt = bt_ref[0, 0, :]
        onehot = (bt[:, None] ==
                  lax.broadcasted_iota(I32, (1, NB_BATCH), 1)).astype(F32)
        gd[...] += lax.dot_general(onehot, ge, (((0,), (0,)), ((), ())),
                                   preferred_element_type=F32)
        pnum[...] += lax.dot_general(onehot, ge * h2,
                                     (((0,), (0,)), ((), ())),
                                     preferred_element_type=F32)

        @pl.when(i == nsteps - 1)
        def _():
            out_ref[...] = pnum[...] / (gd[...] + 1e-16)

    return pl.pallas_call(
        body,
        grid=(nsteps,),
        in_specs=[pl.BlockSpec((2, NF, BN, FW), lambda i: (0, 0, i, 0)),
                  pl.BlockSpec((1, 1, BN), lambda i: (i, 0, 0)),
                  pl.BlockSpec((1, D), lambda i: (0, 0)),
                  pl.BlockSpec((1, 1), lambda i: (0, 0)),
                  pl.BlockSpec((D, D), lambda i: (0, 0)),
                  pl.BlockSpec((1, D), lambda i: (0, 0)),
                  pl.BlockSpec((1, 1), lambda i: (0, 0)),
                  pl.BlockSpec((D, D), lambda i: (0, 0)),
                  pl.BlockSpec((1, D), lambda i: (0, 0)),
                  pl.BlockSpec((1, 1), lambda i: (0, 0)),
                  pl.BlockSpec((D, 128), lambda i: (0, 0)),
                  pl.BlockSpec((1, 1), lambda i: (0, 0))],
        out_specs=pl.BlockSpec((NB_BATCH, D), lambda i: (0, 0)),
        out_shape=jax.ShapeDtypeStruct((NB_BATCH, D), F32),
        scratch_shapes=[pltpu.VMEM((NB_BATCH, D), F32),
                        pltpu.VMEM((NB_BATCH, 1), F32)],
    )(o, batch3, b2, p2, G1, gb1, gp1, G2, gb2, gp2, G3p, gb3)


_sc_mesh = plsc.VectorSubcoreMesh(core_axis_name="c", subcore_axis_name="s")


@functools.partial(
    pl.kernel,
    out_type=jax.ShapeDtypeStruct((2 * NF * Np, FW), F32),
    mesh=_sc_mesh,
    compiler_params=pltpu.CompilerParams(use_tc_tiling_on_sc=False,
                                         needs_layout_passes=False),
    scratch_types=[
        pltpu.VMEM((Np,), F32),
        pltpu.VMEM((Np,), F32),
        pltpu.VMEM((NKC, KC), I32),
        pltpu.VMEM((NKC, KC), I32),
        pltpu.VMEM((EPT,), F32),
        pltpu.VMEM((EPT,), F32),
        pltpu.VMEM((KC, FW), F32),
        pltpu.VMEM((KC, FW), F32),
        pltpu.VMEM((KC, 16), F32),
        pltpu.VMEM((64, FW), F32),
        pltpu.VMEM((64, 16), F32),
        pltpu.VMEM_SHARED((Np, FW), F32),
        pltpu.VMEM_SHARED((Np, 16), F32),
        pltpu.SemaphoreType.DMA,
        pltpu.SemaphoreType.DMA,
    ],
)
def _sc_gat(ss_hbm, sd_hbm, se_hbm, src_hbm, dst_hbm, xp_hbm, out_hbm,
            ssv, sdv, srcv, didxv, sev, wv, rowsa, rowsb, wbuf,
            tmp, tmpd, accsh, densh, sema, semb):
    c = lax.axis_index("c")
    s = lax.axis_index("s")
    ii = lax.iota(I32, 16)
    npt = Np // NT

    pltpu.sync_copy(ss_hbm.at[pl.ds(c * Np, Np)], ssv)
    pltpu.sync_copy(sd_hbm.at[pl.ds(c * Np, Np)], sdv)
    pltpu.sync_copy(src_hbm.at[s], srcv)
    pltpu.sync_copy(dst_hbm.at[s], didxv)
    pltpu.sync_copy(se_hbm.at[pl.ds(c * EE + s * EPT, EPT)], sev)

    def logit_body(j, _):
        for k in range(KC // 16):
            sl = pl.ds(k * 16, 16)
            sr = srcv[j, sl]
            dt = didxv[j, sl]
            a = (plsc.load_gather(ssv, [sr]) + plsc.load_gather(sdv, [dt])
                 + sev[pl.ds(j * KC + k * 16, 16)])
            a = jnp.where(a >= 0, a, 0.2 * a)
            wv[pl.ds(j * KC + k * 16, 16)] = jnp.exp(a)
            srcv[j, sl] = sr + c * (NF * Np)
        return 0
    lax.fori_loop(0, NKC, logit_body, 0)

    zz = jnp.zeros((16,), I32)

    def pass_body(p, _):
        @pl.when(p > 0)
        def _():
            def adv_body(j, _):
                for k in range(KC // 16):
                    sl = pl.ds(k * 16, 16)
                    srcv[j, sl] = srcv[j, sl] + Np
                return 0
            lax.fori_loop(0, NKC, adv_body, 0)

        z16 = jnp.zeros((16,), F32)

        def ztmp_body(r, _):
            for q in range(FW // 16):
                tmp[r, pl.ds(q * 16, 16)] = z16
            tmpd[r] = z16
            return 0
        lax.fori_loop(0, 64, ztmp_body, 0)

        def zero_body(k, _):
            pltpu.sync_copy(tmp, accsh.at[pl.ds(s * npt + k * 64, 64)])
            return 0
        lax.fori_loop(0, npt // 64, zero_body, 0)

        @pl.when(p == 0)
        def _():
            def zden_body(k, _):
                pltpu.sync_copy(tmpd, densh.at[pl.ds(s * npt + k * 64, 64)])
                return 0
            lax.fori_loop(0, npt // 64, zden_body, 0)
        plsc.subcore_barrier()

        def process(j, rows):
            def row_body(r, _):
                av = plsc.load_gather(wv, [jnp.full((16,), j * KC + r, I32)])
                wbuf[r] = av
                for q in range(FW // 16):
                    sl = pl.ds(q * 16, 16)
                    rows[r, sl] = rows[r, sl] * av
                return 0
            lax.fori_loop(0, KC, row_body, 0)
            pltpu.sync_copy(rows, accsh.at[didxv.at[j]], add=True)

            @pl.when(p == 0)
            def _():
                pltpu.sync_copy(wbuf, densh.at[didxv.at[j]], add=True)

        def start_gather(j, rows, sem):
            pltpu.async_copy(xp_hbm.at[srcv.at[j]], rows, sem)

        def wait_gather(rows, sem):
            pltpu.make_async_copy(xp_hbm.at[srcv.at[0]], rows, sem).wait()

        start_gather(0, rowsa, sema)

        def pair_body(t, _):
            j0 = 2 * t
            j1 = j0 + 1
            start_gather(j1, rowsb, semb)
            wait_gather(rowsa, sema)
            process(j0, rowsa)

            @pl.when(j0 + 2 < NKC)
            def _():
                start_gather(j0 + 2, rowsa, sema)
            wait_gather(rowsb, semb)
            process(j1, rowsb)
            return 0
        lax.fori_loop(0, (NKC - 1) // 2, pair_body, 0)
        wait_gather(rowsa, sema)
        process(NKC - 1, rowsa)
        plsc.subcore_barrier()

        obase = (c * NF + p) * Np + s * npt

        def co_body(k, _):
            sl = pl.ds(s * npt + k * 64, 64)
            pltpu.sync_copy(accsh.at[sl], tmp)
            pltpu.sync_copy(densh.at[sl], tmpd)
            for g in range(4):
                denv = plsc.load_gather(tmpd, [ii + g * 16, zz])
                wbuf[g] = 1.0 / (denv + 1e-16)

            def div_body(r, _):
                rcp = plsc.load_gather(
                    wbuf, [jnp.full((16,), r >> 4, I32),
                           jnp.full((16,), r & 15, I32)])
                for q in range(FW // 16):
                    qsl = pl.ds(q * 16, 16)
                    tmp[r, qsl] = tmp[r, qsl] * rcp
                return 0
            lax.fori_loop(0, 64, div_body, 0)
            pltpu.sync_copy(tmp, out_hbm.at[pl.ds(obase + k * 64, 64)])
            return 0
        lax.fori_loop(0, npt // 64, co_body, 0)
        plsc.subcore_barrier()
        return 0

    lax.fori_loop(0, NF, pass_body, 0)


def kernel(x, edge_index, edge_attr, batch,
           W1, We1, as1, ad1, ae1, b1, p1,
           W2, We2, as2, ad2, ae2, b2, p2,
           G1, gb1, gp1, G2, gb2, gp2, G3, gb3):
    src = edge_index[0]
    dst = edge_index[1]
    src3 = src.reshape(NT, NKC, KC)
    dst3 = dst.reshape(NT, NKC, KC)

    xpad = jnp.pad(x, ((0, Np - NN), (0, 0)))
    xpT1, ss1, sd1 = _node_prep(xpad, W1, as1, ad1, 128)
    se1, se2 = _edge_prep(edge_attr, We1, ae1, We2, ae2)

    out1 = _sc_gat(ss1.reshape(-1), sd1.reshape(-1), se1.reshape(-1),
                   src3, dst3, xpT1.reshape(2 * NF * Np, FW))

    xpT2, ss2, sd2 = _mid_prep(out1.reshape(2, NF, Np, FW),
                               b1.reshape(1, D), p1.reshape(1, 1),
                               W2, as2, ad2)

    out2 = _sc_gat(ss2.reshape(-1), sd2.reshape(-1), se2.reshape(-1),
                   src3, dst3, xpT2.reshape(2 * NF * Np, FW))

    batch3 = jnp.pad(batch, (0, Np - NN),
                     constant_values=NB_BATCH).reshape(Np // BN, 1, BN)
    G3p = jnp.pad(G3, ((0, 0), (0, 127)))
    pooled = _final_pool(out2.reshape(2, NF, Np, FW),
                         b2.reshape(1, D), p2.reshape(1, 1),
                         G1, gb1.reshape(1, D), gp1.reshape(1, 1),
                         G2, gb2.reshape(1, D), gp2.reshape(1, 1),
                         G3p, gb3.reshape(1, 1), batch3)
    return pooled.reshape(NB_BATCH, 2, 128)
